# Initial kernel scaffold; baseline (speedup 1.0000x reference)
import jax
import jax.numpy as jnp
from jax.experimental import pallas as pl


def kernel(inputs, table):
    raise NotImplementedError("write your pallas kernel here")


# speedup vs baseline: 1.1300x; 1.1300x over previous
import functools

import jax
import jax.numpy as jnp
from jax import lax
from jax.experimental import pallas as pl
from jax.experimental.pallas import tpu as pltpu
from jax.experimental.pallas import tpu_sc as plsc

_B = 16384
_F = 26
_D = 16
_CB = 128
_IDX_ROWS = _CB * _F // 128


def _make_kernel():
    info = plsc.get_sparse_core_info()
    nc, ns = info.num_cores, info.num_subcores
    nw = nc * ns
    b_per_w = _B // nw
    n_chunks = b_per_w // _CB
    idx_rows_per_w = b_per_w * _F // 128

    mesh = plsc.VectorSubcoreMesh(core_axis_name="c", subcore_axis_name="s")

    @functools.partial(
        pl.kernel,
        mesh=mesh,
        out_type=jax.ShapeDtypeStruct((_B, _D), jnp.float32),
        compiler_params=pltpu.CompilerParams(use_tc_tiling_on_sc=False),
        scratch_types=[
            pltpu.VMEM((idx_rows_per_w, 128), jnp.int32),
            pltpu.VMEM((_CB * _F, _D), jnp.float32),
            pltpu.VMEM((_CB, _D), jnp.float32),
            pltpu.SemaphoreType.DMA,
        ],
    )
    def emb_sum(idx_hbm, table_hbm, out_hbm, idx_v, rows_v, out_v, sem):
        wid = lax.axis_index("s") * nc + lax.axis_index("c")
        pltpu.sync_copy(idx_hbm.at[pl.ds(wid * idx_rows_per_w, idx_rows_per_w)], idx_v)

        def chunk_body(c, carry):
            copies = [
                pltpu.async_copy(
                    table_hbm.at[idx_v.at[c * _IDX_ROWS + k]],
                    rows_v.at[pl.ds(k * 128, 128)],
                    sem,
                )
                for k in range(_IDX_ROWS)
            ]
            for cp in copies:
                cp.wait()

            def reduce_body(i, inner):
                base = i * _F
                acc = rows_v[base]
                for f in range(1, _F):
                    acc = acc + rows_v[base + f]
                out_v[i] = acc
                return inner

            lax.fori_loop(0, _CB, reduce_body, 0)
            pltpu.sync_copy(out_v, out_hbm.at[pl.ds(wid * b_per_w + c * _CB, _CB)])
            return carry

        lax.fori_loop(0, n_chunks, chunk_body, 0)

    return emb_sum


def kernel(inputs, table):
    idx = inputs.reshape(_B * _F).astype(jnp.int32).reshape(_B * _F // 128, 128)
    return _make_kernel()(idx, table)
